# baseline (speedup 1.0000x reference)
import functools

import jax
import jax.numpy as jnp
from jax import lax
from jax.experimental import pallas as pl
from jax.experimental.pallas import tpu as pltpu
from jax.experimental.pallas import tpu_sc as plsc

N = 100000
E = 1600000
F_IN = 38
H1, H2, H3 = 64, 32, 16
G = 64

NP = 100352
EP = 1605632
ER = EP // 128
BLK = 1024
NBLK = NP // BLK


def _make_segsum(*, wh, edge_split, n_pad, er, rb, zr, dtype=jnp.float32):
    n_workers = 32 if edge_split else 16
    rows_w = er // n_workers
    iters = rows_w // rb
    n_chunks = er // rb
    assert iters % 2 == 0 and rows_w % rb == 0
    sub_rows = n_pad // 16
    zfull = sub_rows // zr
    zrem = sub_rows % zr
    mesh = plsc.VectorSubcoreMesh(core_axis_name="c", subcore_axis_name="s")

    @functools.partial(
        pl.kernel,
        out_type=(jax.ShapeDtypeStruct((n_pad, wh), dtype),
                  jax.ShapeDtypeStruct((n_pad, wh), dtype)),
        mesh=mesh,
        scratch_types=[
            pltpu.VMEM((2, 2, rb * 128), jnp.int32),
            pltpu.VMEM((2, rb * 128, wh), dtype),
            pltpu.VMEM((zr, wh), dtype),
            pltpu.VMEM_SHARED((n_pad, wh), dtype),
            pltpu.SemaphoreType.DMA,
            pltpu.SemaphoreType.DMA,
            pltpu.SemaphoreType.DMA,
        ],
        compiler_params=pltpu.CompilerParams(use_tc_tiling_on_sc=False),
    )
    def segsum(ta, tb, packed, z, out_a, out_b,
               idx_v, rows_v, zbuf, acc, isem, gsem, ssem):
        cid = lax.axis_index("c")
        sid = lax.axis_index("s")

        pltpu.sync_copy(z, zbuf)
        for k in range(zfull):
            pltpu.sync_copy(zbuf,
                            acc.at[pl.ds(sid * sub_rows + k * zr, zr)])
        if zrem:
            pltpu.sync_copy(zbuf.at[pl.ds(0, zrem)],
                            acc.at[pl.ds(sid * sub_rows + zfull * zr, zrem)])
        plsc.subcore_barrier()

        if edge_split:
            cbase = (cid * 16 + sid) * iters
        else:
            cbase = sid * iters

        def run(table):
            def drain_idx(p):
                pltpu.make_async_copy(packed.at[0], idx_v.at[p], isem).wait()

            def drain_rows(sem, p):
                pltpu.make_async_copy(z, rows_v.at[p], sem).wait()

            def fire_idx(p, it):
                pltpu.async_copy(packed.at[cbase + it], idx_v.at[p], isem)

            def half(i2, p):
                it = i2 * 2 + p
                drain_idx(p)
                pltpu.async_copy(table.at[idx_v.at[p, 0]],
                                 rows_v.at[p], gsem)
                drain_rows(gsem, p)

                @pl.when(it >= 1)
                def _():
                    drain_rows(ssem, 1 - p)

                @pl.when(it + 1 < iters)
                def _():
                    fire_idx(1 - p, it + 1)
                pltpu.async_copy(rows_v.at[p], acc.at[idx_v.at[p, 1]],
                                 ssem, add=True)

            def body(i2, carry):
                half(i2, 0)
                half(i2, 1)
                return carry

            fire_idx(0, 0)
            lax.fori_loop(0, iters // 2, body, 0)
            drain_rows(ssem, 1)

        @pl.when(cid == 0)
        def _():
            run(ta)

        @pl.when(cid == 1)
        def _():
            run(tb)

        plsc.subcore_barrier()

        @pl.when(cid == 0)
        def _():
            pltpu.sync_copy(acc.at[pl.ds(sid * sub_rows, sub_rows)],
                            out_a.at[pl.ds(sid * sub_rows, sub_rows)])

        @pl.when(cid == 1)
        def _():
            pltpu.sync_copy(acc.at[pl.ds(sid * sub_rows, sub_rows)],
                            out_b.at[pl.ds(sid * sub_rows, sub_rows)])

    return segsum


def _dotT(a, b):
    return lax.dot_general(a, b, (((0,), (0,)), ((), ())),
                           preferred_element_type=jnp.float32)


def _dot(a, b):
    return jnp.dot(a, b, preferred_element_type=jnp.float32)


def _pack_body(ei_ref, out_ref):
    gid = pl.program_id(0)
    ei = ei_ref[...]
    i0 = lax.broadcasted_iota(jnp.int32, (32, 512), 0)
    i1 = lax.broadcasted_iota(jnp.int32, (32, 512), 1)
    flat = gid * 16384 + i0 * 512 + i1
    valid = flat < E
    padv = N + lax.rem(flat, 256)
    src = jnp.where(valid, ei[0].reshape(32, 512), padv)
    dst = jnp.where(valid, ei[1].reshape(32, 512), padv)
    out_ref[...] = jnp.concatenate([src[:, None], dst[:, None]], axis=1)


def _tc1_body(agg1a, agg1b, agg8a, agg8b, x,
              w1rel, w1root, b1, w2rel, h1_ref, p2_ref):
    wr = w1rel[...]
    agg1 = (agg1a[...].astype(jnp.float32)
            + agg1b[...].astype(jnp.float32))
    a = (_dot(agg1, wr[:32]) + _dot(agg8a[...] + agg8b[...], wr[32:])
         + _dot(x[...], w1root[...]) + b1[...])
    h1 = jnp.tanh(a)
    h1_ref[...] = h1
    p2_ref[...] = _dot(h1, w2rel[...]).astype(jnp.bfloat16)


def _tc2_body(agg2a, agg2b, h1, w2root, b2, w3rel, h2_ref, p3_ref):
    a = agg2a[...].astype(jnp.float32) + agg2b[...].astype(jnp.float32)
    h2 = jnp.tanh(a + _dot(h1[...], w2root[...]) + b2[...])
    h2_ref[...] = h2
    p3_ref[...] = _dot(h2, w3rel[...])


def _tc3_body(agg_a, agg_b, h2, w3root, b3, batch, out_ref, sums, cnts):
    i = pl.program_id(0)

    @pl.when(i == 0)
    def _():
        sums[...] = jnp.zeros_like(sums)
        cnts[...] = jnp.zeros_like(cnts)

    h3 = agg_a[...] + agg_b[...] + _dot(h2[...], w3root[...]) + b3[...]
    h3 = jnp.where(batch[...] < G, h3, 0.0)
    oh = (batch[...] == lax.broadcasted_iota(jnp.int32, (BLK, G), 1))
    oh = oh.astype(jnp.float32)
    sums[...] += _dotT(oh, h3)
    cnts[...] += _dotT(oh, jnp.ones((BLK, H3), jnp.float32))

    @pl.when(i == NBLK - 1)
    def _():
        out_ref[...] = jnp.tanh(sums[...] / jnp.maximum(cnts[...], 1.0))


def _row_spec(w):
    return pl.BlockSpec((BLK, w), lambda i: (i, 0))


def _full_spec(r, c):
    return pl.BlockSpec((r, c), lambda i: (0, 0))


def kernel(x, edge_index, batch, W1_rel, b1, W1_root,
           W2_rel, b2, W2_root, W3_rel, b3, W3_root):
    f32 = jnp.float32

    pad_ids = N + (jnp.arange(EP - E, dtype=jnp.int32) % 256)
    packed = jnp.stack(
        [jnp.concatenate([edge_index[0], pad_ids]).reshape(ER // 4, 512),
         jnp.concatenate([edge_index[1], pad_ids]).reshape(ER // 4, 512)],
        axis=1)

    bf16 = jnp.bfloat16
    xbf = jnp.zeros((NP, 32), bf16).at[:N].set(x[:, :32].astype(bf16))
    x8 = jnp.zeros((NP, 8), f32).at[:N, :6].set(x[:, 32:])
    w1rel = jnp.zeros((40, H1), f32).at[:F_IN].set(W1_rel)
    z16 = jnp.zeros((512, 16), f32)
    z8 = jnp.zeros((512, 8), f32)
    zb32 = jnp.zeros((512, 32), bf16)
    batchp = jnp.concatenate(
        [batch, jnp.full((NP - N,), G, jnp.int32)]).reshape(NP, 1)

    seg32b = _make_segsum(wh=32, edge_split=True, n_pad=NP, er=ER, rb=4,
                          zr=512, dtype=bf16)
    seg8e = _make_segsum(wh=8, edge_split=True, n_pad=NP, er=ER, rb=4,
                         zr=512)
    seg16e = _make_segsum(wh=16, edge_split=True, n_pad=NP, er=ER, rb=4,
                          zr=512)

    agg1a, agg1b = seg32b(xbf, xbf, packed, zb32)
    agg8a, agg8b = seg8e(x8, x8, packed, z8)

    h1, p2 = pl.pallas_call(
        _tc1_body,
        grid=(NBLK,),
        in_specs=[_row_spec(32), _row_spec(32), _row_spec(8), _row_spec(8),
                  pl.BlockSpec((BLK, F_IN), lambda i: (i, 0)),
                  _full_spec(40, H1), _full_spec(F_IN, H1), _full_spec(1, H1),
                  _full_spec(H1, H2)],
        out_specs=[_row_spec(H1), _row_spec(H2)],
        out_shape=[jax.ShapeDtypeStruct((NP, H1), f32),
                   jax.ShapeDtypeStruct((NP, H2), jnp.bfloat16)],
    )(agg1a, agg1b, agg8a, agg8b, x, w1rel, W1_root,
      b1.reshape(1, H1), W2_rel)

    agg2a, agg2b = seg32b(p2, p2, packed, zb32)

    h2, p3 = pl.pallas_call(
        _tc2_body,
        grid=(NBLK,),
        in_specs=[_row_spec(H2), _row_spec(H2), _row_spec(H1),
                  _full_spec(H1, H2), _full_spec(1, H2), _full_spec(H2, H3)],
        out_specs=[_row_spec(H2), _row_spec(H3)],
        out_shape=[jax.ShapeDtypeStruct((NP, H2), f32),
                   jax.ShapeDtypeStruct((NP, H3), f32)],
    )(agg2a, agg2b, h1, W2_root, b2.reshape(1, H2), W3_rel)

    agg3_a, agg3_b = seg16e(p3, p3, packed, z16)

    out = pl.pallas_call(
        _tc3_body,
        grid=(NBLK,),
        in_specs=[_row_spec(H3), _row_spec(H3), _row_spec(H2),
                  _full_spec(H2, H3), _full_spec(1, H3),
                  pl.BlockSpec((BLK, 1), lambda i: (i, 0))],
        out_specs=pl.BlockSpec((G, H3), lambda i: (0, 0)),
        out_shape=jax.ShapeDtypeStruct((G, H3), f32),
        scratch_shapes=[pltpu.VMEM((G, H3), f32), pltpu.VMEM((G, H3), f32)],
    )(agg3_a, agg3_b, h2, W3_root, b3.reshape(1, H3), batchp)

    return out


# speedup vs baseline: 1.0139x; 1.0139x over previous
import functools

import jax
import jax.numpy as jnp
from jax import lax
from jax.experimental import pallas as pl
from jax.experimental.pallas import tpu as pltpu
from jax.experimental.pallas import tpu_sc as plsc

N = 100000
E = 1600000
F_IN = 38
H1, H2, H3 = 64, 32, 16
G = 64

NP = 100352
EP = 1605632
ER = EP // 128
BLK = 1024
NBLK = NP // BLK


def _make_segsum(*, wh, edge_split, n_pad, er, rb, zr, dtype=jnp.float32):
    n_workers = 32 if edge_split else 16
    rows_w = er // n_workers
    iters = rows_w // rb
    n_chunks = er // rb
    assert iters % 2 == 0 and rows_w % rb == 0
    sub_rows = n_pad // 16
    zfull = sub_rows // zr
    zrem = sub_rows % zr
    mesh = plsc.VectorSubcoreMesh(core_axis_name="c", subcore_axis_name="s")

    @functools.partial(
        pl.kernel,
        out_type=(jax.ShapeDtypeStruct((n_pad, wh), dtype),
                  jax.ShapeDtypeStruct((n_pad, wh), dtype)),
        mesh=mesh,
        scratch_types=[
            pltpu.VMEM((2, 2, rb * 128), jnp.int32),
            pltpu.VMEM((2, rb * 128, wh), dtype),
            pltpu.VMEM((zr, wh), dtype),
            pltpu.VMEM_SHARED((n_pad, wh), dtype),
            pltpu.SemaphoreType.DMA,
            pltpu.SemaphoreType.DMA,
            pltpu.SemaphoreType.DMA,
        ],
        compiler_params=pltpu.CompilerParams(use_tc_tiling_on_sc=False),
    )
    def segsum(ta, tb, packed, z, out_a, out_b,
               idx_v, rows_v, zbuf, acc, isem, gsem, ssem):
        cid = lax.axis_index("c")
        sid = lax.axis_index("s")

        pltpu.sync_copy(z, zbuf)
        for k in range(zfull):
            pltpu.sync_copy(zbuf,
                            acc.at[pl.ds(sid * sub_rows + k * zr, zr)])
        if zrem:
            pltpu.sync_copy(zbuf.at[pl.ds(0, zrem)],
                            acc.at[pl.ds(sid * sub_rows + zfull * zr, zrem)])
        plsc.subcore_barrier()

        if edge_split:
            cbase = (cid * 16 + sid) * iters
        else:
            cbase = sid * iters

        def run(table):
            def drain_idx(p):
                pltpu.make_async_copy(packed.at[0], idx_v.at[p], isem).wait()

            def drain_rows(sem, p):
                pltpu.make_async_copy(z, rows_v.at[p], sem).wait()

            def fire_idx(p, it):
                pltpu.async_copy(packed.at[cbase + it], idx_v.at[p], isem)

            def half(i2, p):
                it = i2 * 2 + p
                drain_idx(p)
                pltpu.async_copy(table.at[idx_v.at[p, 0]],
                                 rows_v.at[p], gsem)
                drain_rows(gsem, p)

                @pl.when(it >= 1)
                def _():
                    drain_rows(ssem, 1 - p)

                @pl.when(it + 1 < iters)
                def _():
                    fire_idx(1 - p, it + 1)
                pltpu.async_copy(rows_v.at[p], acc.at[idx_v.at[p, 1]],
                                 ssem, add=True)

            def body(i2, carry):
                half(i2, 0)
                half(i2, 1)
                return carry

            fire_idx(0, 0)
            lax.fori_loop(0, iters // 2, body, 0)
            drain_rows(ssem, 1)

        @pl.when(cid == 0)
        def _():
            run(ta)

        @pl.when(cid == 1)
        def _():
            run(tb)

        plsc.subcore_barrier()

        @pl.when(cid == 0)
        def _():
            pltpu.sync_copy(acc.at[pl.ds(sid * sub_rows, sub_rows)],
                            out_a.at[pl.ds(sid * sub_rows, sub_rows)])

        @pl.when(cid == 1)
        def _():
            pltpu.sync_copy(acc.at[pl.ds(sid * sub_rows, sub_rows)],
                            out_b.at[pl.ds(sid * sub_rows, sub_rows)])

    return segsum


def _dotT(a, b):
    return lax.dot_general(a, b, (((0,), (0,)), ((), ())),
                           preferred_element_type=jnp.float32)


def _dot(a, b):
    return jnp.dot(a, b, preferred_element_type=jnp.float32)


def _pack_body(ei_ref, out_ref):
    gid = pl.program_id(0)
    ei = ei_ref[...]
    i0 = lax.broadcasted_iota(jnp.int32, (32, 512), 0)
    i1 = lax.broadcasted_iota(jnp.int32, (32, 512), 1)
    flat = gid * 16384 + i0 * 512 + i1
    valid = flat < E
    padv = N + lax.rem(flat, 256)
    src = jnp.where(valid, ei[0].reshape(32, 512), padv)
    dst = jnp.where(valid, ei[1].reshape(32, 512), padv)
    out_ref[...] = jnp.concatenate([src[:, None], dst[:, None]], axis=1)


def _tc1_body(agg1a, agg1b, agg8a, agg8b, x,
              w1rel, w1root, b1, w2rel, h1_ref, p2_ref):
    wr = w1rel[...]
    agg1 = (agg1a[...].astype(jnp.float32)
            + agg1b[...].astype(jnp.float32))
    a = (_dot(agg1, wr[:32]) + _dot(agg8a[...] + agg8b[...], wr[32:])
         + _dot(x[...], w1root[...]) + b1[...])
    h1 = jnp.tanh(a)
    h1_ref[...] = h1
    p2_ref[...] = _dot(h1, w2rel[...]).astype(jnp.bfloat16)


def _tc2_body(agg2a, agg2b, h1, w2root, b2, w3rel, h2_ref, p3_ref):
    a = agg2a[...].astype(jnp.float32) + agg2b[...].astype(jnp.float32)
    h2 = jnp.tanh(a + _dot(h1[...], w2root[...]) + b2[...])
    h2_ref[...] = h2
    p3_ref[...] = _dot(h2, w3rel[...])


def _tc3_body(agg_a, agg_b, h2, w3root, b3, batch, out_ref, sums, cnts):
    i = pl.program_id(0)

    @pl.when(i == 0)
    def _():
        sums[...] = jnp.zeros_like(sums)
        cnts[...] = jnp.zeros_like(cnts)

    h3 = agg_a[...] + agg_b[...] + _dot(h2[...], w3root[...]) + b3[...]
    h3 = jnp.where(batch[...] < G, h3, 0.0)
    oh = (batch[...] == lax.broadcasted_iota(jnp.int32, (BLK, G), 1))
    oh = oh.astype(jnp.float32)
    sums[...] += _dotT(oh, h3)
    cnts[...] += _dotT(oh, jnp.ones((BLK, H3), jnp.float32))

    @pl.when(i == NBLK - 1)
    def _():
        out_ref[...] = jnp.tanh(sums[...] / jnp.maximum(cnts[...], 1.0))


def _row_spec(w):
    return pl.BlockSpec((BLK, w), lambda i: (i, 0))


def _full_spec(r, c):
    return pl.BlockSpec((r, c), lambda i: (0, 0))


def kernel(x, edge_index, batch, W1_rel, b1, W1_root,
           W2_rel, b2, W2_root, W3_rel, b3, W3_root):
    f32 = jnp.float32

    packed = pl.pallas_call(
        _pack_body,
        grid=(NBLK,),
        in_specs=[pl.BlockSpec((2, 16384), lambda i: (0, i))],
        out_specs=pl.BlockSpec((32, 2, 512), lambda i: (i, 0, 0)),
        out_shape=jax.ShapeDtypeStruct((ER // 4, 2, 512), jnp.int32),
    )(edge_index)

    bf16 = jnp.bfloat16
    xbf = jnp.zeros((NP, 32), bf16).at[:N].set(x[:, :32].astype(bf16))
    x8 = jnp.zeros((NP, 8), f32).at[:N, :6].set(x[:, 32:])
    w1rel = jnp.zeros((40, H1), f32).at[:F_IN].set(W1_rel)
    z16 = jnp.zeros((512, 16), f32)
    z8 = jnp.zeros((512, 8), f32)
    zb32 = jnp.zeros((512, 32), bf16)
    batchp = jnp.concatenate(
        [batch, jnp.full((NP - N,), G, jnp.int32)]).reshape(NP, 1)

    seg32b = _make_segsum(wh=32, edge_split=True, n_pad=NP, er=ER, rb=4,
                          zr=512, dtype=bf16)
    seg8e = _make_segsum(wh=8, edge_split=True, n_pad=NP, er=ER, rb=4,
                         zr=512)
    seg16e = _make_segsum(wh=16, edge_split=True, n_pad=NP, er=ER, rb=4,
                          zr=512)

    agg1a, agg1b = seg32b(xbf, xbf, packed, zb32)
    agg8a, agg8b = seg8e(x8, x8, packed, z8)

    h1, p2 = pl.pallas_call(
        _tc1_body,
        grid=(NBLK,),
        in_specs=[_row_spec(32), _row_spec(32), _row_spec(8), _row_spec(8),
                  pl.BlockSpec((BLK, F_IN), lambda i: (i, 0)),
                  _full_spec(40, H1), _full_spec(F_IN, H1), _full_spec(1, H1),
                  _full_spec(H1, H2)],
        out_specs=[_row_spec(H1), _row_spec(H2)],
        out_shape=[jax.ShapeDtypeStruct((NP, H1), f32),
                   jax.ShapeDtypeStruct((NP, H2), jnp.bfloat16)],
    )(agg1a, agg1b, agg8a, agg8b, x, w1rel, W1_root,
      b1.reshape(1, H1), W2_rel)

    agg2a, agg2b = seg32b(p2, p2, packed, zb32)

    h2, p3 = pl.pallas_call(
        _tc2_body,
        grid=(NBLK,),
        in_specs=[_row_spec(H2), _row_spec(H2), _row_spec(H1),
                  _full_spec(H1, H2), _full_spec(1, H2), _full_spec(H2, H3)],
        out_specs=[_row_spec(H2), _row_spec(H3)],
        out_shape=[jax.ShapeDtypeStruct((NP, H2), f32),
                   jax.ShapeDtypeStruct((NP, H3), f32)],
    )(agg2a, agg2b, h1, W2_root, b2.reshape(1, H2), W3_rel)

    agg3_a, agg3_b = seg16e(p3, p3, packed, z16)

    out = pl.pallas_call(
        _tc3_body,
        grid=(NBLK,),
        in_specs=[_row_spec(H3), _row_spec(H3), _row_spec(H2),
                  _full_spec(H2, H3), _full_spec(1, H3),
                  pl.BlockSpec((BLK, 1), lambda i: (i, 0))],
        out_specs=pl.BlockSpec((G, H3), lambda i: (0, 0)),
        out_shape=jax.ShapeDtypeStruct((G, H3), f32),
        scratch_shapes=[pltpu.VMEM((G, H3), f32), pltpu.VMEM((G, H3), f32)],
    )(agg3_a, agg3_b, h2, W3_root, b3.reshape(1, H3), batchp)

    return out
